# baseline (speedup 1.0000x reference)
import functools

import jax
import jax.numpy as jnp
from jax import lax
from jax.experimental import pallas as pl
from jax.experimental.pallas import tpu as pltpu
from jax.experimental.pallas import tpu_sc as plsc

_NUM_KPS = 17
_POS_DISTANCE = 4.0
_ROI_EXPAND = 1.2
_FW = 56
_FH = 56
_HW = _FW * _FH
_NREG_CH = 2 * _NUM_KPS


def _argmax_body(rois_ref, s_ref, ms_ref, bx_ref, by_ref, scx_ref, scy_ref,
                 idx_ref):
    i = pl.program_id(0)
    s = s_ref[...]
    blk = s.shape[0]
    m = jnp.max(s, axis=(-2, -1))
    ir = lax.broadcasted_iota(jnp.int32, s.shape, 2)
    ic = lax.broadcasted_iota(jnp.int32, s.shape, 3)
    fid = ir * _FW + ic
    fidx = jnp.min(jnp.where(s == m[..., None, None], fid, _HW), axis=(-2, -1))
    fidxf = fidx.astype(jnp.float32)
    iy = jnp.floor(fidxf / _FW)
    ix = fidxf - iy * _FW

    r = rois_ref[...]
    w = (r[:, 2] - r[:, 0]) * _ROI_EXPAND
    h = (r[:, 3] - r[:, 1]) * _ROI_EXPAND
    x1 = (r[:, 2] + r[:, 0]) * 0.5 - w * 0.5
    y1 = (r[:, 3] + r[:, 1]) * 0.5 - h * 0.5
    sx = _FW / (w + 1.0)
    sy = _FW / (h + 1.0)

    g = i * blk + lax.broadcasted_iota(jnp.int32, fidx.shape, 0)
    k = lax.broadcasted_iota(jnp.int32, fidx.shape, 1)
    idx_ref[...] = (g * _NREG_CH + 2 * k) * _HW + fidx

    ms_ref[...] = m
    bx_ref[...] = ix / sx[:, None] + x1[:, None]
    by_ref[...] = iy / sy[:, None] + y1[:, None]
    scx_ref[...] = jnp.broadcast_to((_POS_DISTANCE / sx)[:, None], fidx.shape)
    scy_ref[...] = jnp.broadcast_to((_POS_DISTANCE / sy)[:, None], fidx.shape)


def _tc_argmax(rois, scores, n, block):
    grid = (n // block,)
    out2d_f = jax.ShapeDtypeStruct((n, _NUM_KPS), jnp.float32)
    out2d_i = jax.ShapeDtypeStruct((n, _NUM_KPS), jnp.int32)
    spec2d = pl.BlockSpec((block, _NUM_KPS), lambda i: (i, 0))
    return pl.pallas_call(
        _argmax_body,
        grid=grid,
        in_specs=[
            pl.BlockSpec((block, 4), lambda i: (i, 0)),
            pl.BlockSpec((block, _NUM_KPS, _FW, _FH), lambda i: (i, 0, 0, 0)),
        ],
        out_specs=[spec2d] * 6,
        out_shape=[out2d_f] * 5 + [out2d_i],
    )(rois, scores)


def _sc_gather(reg_flat, idx_all, base_all, scale_all, total, per_tile):
    mesh = plsc.VectorSubcoreMesh(core_axis_name="c", subcore_axis_name="s")
    n_chunks = per_tile // 16

    @functools.partial(
        pl.kernel,
        out_type=jax.ShapeDtypeStruct((total,), jnp.float32),
        mesh=mesh,
        scratch_types=[
            pltpu.VMEM((per_tile,), jnp.int32),
            pltpu.VMEM((per_tile,), jnp.float32),
            pltpu.VMEM((per_tile,), jnp.float32),
            pltpu.VMEM((per_tile,), jnp.float32),
            pltpu.VMEM((per_tile,), jnp.float32),
            pltpu.SemaphoreType.DMA,
        ],
    )
    def gather_combine(reg_hbm, idx_hbm, b_hbm, s_hbm, out_hbm,
                       idx_v, d_v, b_v, s_v, o_v, sem):
        wid = lax.axis_index("s") * 2 + lax.axis_index("c")
        start = wid * per_tile
        pltpu.sync_copy(idx_hbm.at[pl.ds(start, per_tile)], idx_v)
        cp = pltpu.async_copy(reg_hbm.at[idx_v], d_v, sem)
        pltpu.sync_copy(b_hbm.at[pl.ds(start, per_tile)], b_v)
        pltpu.sync_copy(s_hbm.at[pl.ds(start, per_tile)], s_v)
        cp.wait()

        def body(j, carry):
            sl = pl.ds(j * 16, 16)
            o_v[sl] = b_v[sl] + d_v[sl] * s_v[sl]
            return carry

        lax.fori_loop(0, n_chunks, body, 0)
        pltpu.sync_copy(o_v, out_hbm.at[pl.ds(start, per_tile)])

    return gather_combine(reg_flat, idx_all, base_all, scale_all)


@functools.partial(jax.jit, static_argnames=("block",))
def kernel(batch_rois, kps_rcnn_cls_pred, kps_rcnn_reg_pred, block=8):
    bs, r_per = batch_rois.shape[0], batch_rois.shape[1]
    n = bs * r_per
    scores = kps_rcnn_cls_pred.reshape(n, _NUM_KPS, _FW, _FH)
    reg_flat = kps_rcnn_reg_pred.reshape(n * _NREG_CH * _HW)
    rois = batch_rois[..., :4].reshape(n, 4)

    ms, bx, by, scx, scy, idxx = _tc_argmax(rois, scores, n, block)

    nk = n * _NUM_KPS
    idx_all = jnp.concatenate([idxx.reshape(nk), idxx.reshape(nk) + _HW])
    base_all = jnp.concatenate([bx.reshape(nk), by.reshape(nk)])
    scale_all = jnp.concatenate([scx.reshape(nk), scy.reshape(nk)])

    total = 2 * nk
    per_tile = total // 32
    res = _sc_gather(reg_flat, idx_all, base_all, scale_all, total, per_tile)

    px = res[:nk].reshape(n, _NUM_KPS)
    py = res[nk:].reshape(n, _NUM_KPS)
    return jnp.stack([px, py, ms], axis=-1).reshape(bs, r_per, _NUM_KPS, 3)


# speedup vs baseline: 1.6896x; 1.6896x over previous
import functools

import jax
import jax.numpy as jnp
from jax import lax
from jax.experimental import pallas as pl

_NUM_KPS = 17
_POS_DISTANCE = 4.0
_ROI_EXPAND = 1.2
_FW = 56
_FH = 56
_HW = _FW * _FH


def _decode_body(rois_ref, s_ref, d_ref, px_ref, py_ref, ms_ref):
    s = s_ref[...]
    m = jnp.max(s, axis=(-2, -1))
    ir = lax.broadcasted_iota(jnp.int32, s.shape, 2)
    ic = lax.broadcasted_iota(jnp.int32, s.shape, 3)
    fid = ir * _FW + ic
    fidx = jnp.min(jnp.where(s == m[..., None, None], fid, _HW), axis=(-2, -1))
    onehot = fid == fidx[..., None, None]
    dx = jnp.sum(jnp.where(onehot, d_ref[:, :, 0], 0.0), axis=(-2, -1))
    dy = jnp.sum(jnp.where(onehot, d_ref[:, :, 1], 0.0), axis=(-2, -1))
    fidxf = fidx.astype(jnp.float32)
    iy = jnp.floor(fidxf / _FW)
    ix = fidxf - iy * _FW

    r = rois_ref[...]
    w = (r[:, 2] - r[:, 0]) * _ROI_EXPAND
    h = (r[:, 3] - r[:, 1]) * _ROI_EXPAND
    x1 = (r[:, 2] + r[:, 0]) * 0.5 - w * 0.5
    y1 = (r[:, 3] + r[:, 1]) * 0.5 - h * 0.5
    sx = _FW / (w + 1.0)
    sy = _FW / (h + 1.0)

    px_ref[...] = (ix + dx * _POS_DISTANCE) / sx[:, None] + x1[:, None]
    py_ref[...] = (iy + dy * _POS_DISTANCE) / sy[:, None] + y1[:, None]
    ms_ref[...] = m


@functools.partial(jax.jit, static_argnames=("block",))
def kernel(batch_rois, kps_rcnn_cls_pred, kps_rcnn_reg_pred, block=8):
    bs, r_per = batch_rois.shape[0], batch_rois.shape[1]
    n = bs * r_per
    scores = kps_rcnn_cls_pred.reshape(n, _NUM_KPS, _FW, _FH)
    deltas = kps_rcnn_reg_pred.reshape(n, _NUM_KPS, 2, _FW, _FH)
    rois = batch_rois[..., :4].reshape(n, 4)

    grid = (n // block,)
    spec2d = pl.BlockSpec((block, _NUM_KPS), lambda i: (i, 0))
    out2d = jax.ShapeDtypeStruct((n, _NUM_KPS), jnp.float32)
    px, py, ms = pl.pallas_call(
        _decode_body,
        grid=grid,
        in_specs=[
            pl.BlockSpec((block, 4), lambda i: (i, 0)),
            pl.BlockSpec((block, _NUM_KPS, _FW, _FH), lambda i: (i, 0, 0, 0)),
            pl.BlockSpec((block, _NUM_KPS, 2, _FW, _FH),
                         lambda i: (i, 0, 0, 0, 0)),
        ],
        out_specs=[spec2d] * 3,
        out_shape=[out2d] * 3,
    )(rois, scores, deltas)

    return jnp.stack([px, py, ms], axis=-1).reshape(bs, r_per, _NUM_KPS, 3)


# speedup vs baseline: 1.7312x; 1.0246x over previous
import functools

import jax
import jax.numpy as jnp
from jax import lax
from jax.experimental import pallas as pl

_NUM_KPS = 17
_POS_DISTANCE = 4.0
_ROI_EXPAND = 1.2
_FW = 56
_FH = 56
_HW = _FW * _FH


def _decode_body(rois_ref, s_ref, d_ref, px_ref, py_ref, ms_ref):
    s = s_ref[...]
    m = jnp.max(s, axis=(-2, -1))
    ir = lax.broadcasted_iota(jnp.int32, s.shape, 2)
    ic = lax.broadcasted_iota(jnp.int32, s.shape, 3)
    fid = ir * _FW + ic
    fidx = jnp.min(jnp.where(s == m[..., None, None], fid, _HW), axis=(-2, -1))
    onehot = fid == fidx[..., None, None]
    dx = jnp.sum(jnp.where(onehot, d_ref[:, :, 0], 0.0), axis=(-2, -1))
    dy = jnp.sum(jnp.where(onehot, d_ref[:, :, 1], 0.0), axis=(-2, -1))
    fidxf = fidx.astype(jnp.float32)
    iy = jnp.floor(fidxf / _FW)
    ix = fidxf - iy * _FW

    r = rois_ref[...]
    w = (r[:, 2] - r[:, 0]) * _ROI_EXPAND
    h = (r[:, 3] - r[:, 1]) * _ROI_EXPAND
    x1 = (r[:, 2] + r[:, 0]) * 0.5 - w * 0.5
    y1 = (r[:, 3] + r[:, 1]) * 0.5 - h * 0.5
    sx = _FW / (w + 1.0)
    sy = _FW / (h + 1.0)

    px_ref[...] = (ix + dx * _POS_DISTANCE) / sx[:, None] + x1[:, None]
    py_ref[...] = (iy + dy * _POS_DISTANCE) / sy[:, None] + y1[:, None]
    ms_ref[...] = m


@functools.partial(jax.jit, static_argnames=("block",))
def kernel(batch_rois, kps_rcnn_cls_pred, kps_rcnn_reg_pred, block=16):
    bs, r_per = batch_rois.shape[0], batch_rois.shape[1]
    n = bs * r_per
    scores = kps_rcnn_cls_pred.reshape(n, _NUM_KPS, _FW, _FH)
    deltas = kps_rcnn_reg_pred.reshape(n, _NUM_KPS, 2, _FW, _FH)
    rois = batch_rois[..., :4].reshape(n, 4)

    grid = (n // block,)
    spec2d = pl.BlockSpec((block, _NUM_KPS), lambda i: (i, 0))
    out2d = jax.ShapeDtypeStruct((n, _NUM_KPS), jnp.float32)
    px, py, ms = pl.pallas_call(
        _decode_body,
        grid=grid,
        in_specs=[
            pl.BlockSpec((block, 4), lambda i: (i, 0)),
            pl.BlockSpec((block, _NUM_KPS, _FW, _FH), lambda i: (i, 0, 0, 0)),
            pl.BlockSpec((block, _NUM_KPS, 2, _FW, _FH),
                         lambda i: (i, 0, 0, 0, 0)),
        ],
        out_specs=[spec2d] * 3,
        out_shape=[out2d] * 3,
    )(rois, scores, deltas)

    return jnp.stack([px, py, ms], axis=-1).reshape(bs, r_per, _NUM_KPS, 3)
